# Initial kernel scaffold; baseline (speedup 1.0000x reference)
import jax
import jax.numpy as jnp
from jax.experimental import pallas as pl


def kernel(node_energy, forces, species, local_or_ghost, batch, ptr, enr_table):
    raise NotImplementedError("write your pallas kernel here")


# speedup vs baseline: 19.2353x; 19.2353x over previous
import functools

import jax
import jax.numpy as jnp
from jax import lax
from jax.experimental import pallas as pl
from jax.experimental.pallas import tpu as pltpu
from jax.experimental.pallas import tpu_sc as plsc

N_NODES = 100000
NUM_GRAPHS = 64
NW = 32
CHUNK = 3136
N_PAD = NW * CHUNK
TABLE_PAD = 128
L = 16
VECS = CHUNK // L


def _sc_body(ne_hbm, sp_hbm, lg_hbm, bt_hbm, tab_hbm,
             oe_hbm, par_hbm,
             ne_v, sp_v, lg_v, bt_v, oe_v, tab_v, acc_v, par_v):
    wid = lax.axis_index("s") * 2 + lax.axis_index("c")
    base = wid * CHUNK

    pltpu.sync_copy(tab_hbm, tab_v)
    pltpu.sync_copy(ne_hbm.at[pl.ds(base, CHUNK)], ne_v)
    pltpu.sync_copy(sp_hbm.at[pl.ds(base, CHUNK)], sp_v)
    pltpu.sync_copy(lg_hbm.at[pl.ds(base, CHUNK)], lg_v)
    pltpu.sync_copy(bt_hbm.at[pl.ds(base, CHUNK)], bt_v)

    zeros16 = jnp.zeros((L,), jnp.float32)

    def zero_body(i, _):
        acc_v[pl.ds(i * L, L)] = zeros16
        return _

    lax.fori_loop(0, (L * NUM_GRAPHS) // L, zero_body, None)

    lane_base = lax.iota(jnp.int32, L) * NUM_GRAPHS

    def body(i, _):
        sl = pl.ds(i * L, L)
        idx = sp_v[sl]
        tv = plsc.load_gather(tab_v, [idx])
        lgv = lg_v[sl]
        e = ne_v[sl] + tv * lgv
        oe_v[sl] = e
        loc = e * lgv
        fidx = lane_base + bt_v[sl]
        plsc.addupdate_scatter(acc_v, [fidx], loc)
        return _

    lax.fori_loop(0, VECS, body, None)

    def fold_body(j, _):
        s = jnp.zeros((L,), jnp.float32)
        for l in range(L):
            s = s + acc_v[pl.ds(l * NUM_GRAPHS + j * L, L)]
        par_v[pl.ds(j * L, L)] = s
        return _

    lax.fori_loop(0, NUM_GRAPHS // L, fold_body, None)

    pltpu.sync_copy(oe_v, oe_hbm.at[pl.ds(base, CHUNK)])
    pltpu.sync_copy(par_v, par_hbm.at[wid])


@jax.jit
def _sc_call(ne_p, sp_p, lg_p, bt_p, tab_p):
    mesh = plsc.VectorSubcoreMesh(core_axis_name="c", subcore_axis_name="s")
    k = functools.partial(
        pl.kernel,
        mesh=mesh,
        compiler_params=pltpu.CompilerParams(needs_layout_passes=False),
        out_type=(
            jax.ShapeDtypeStruct((N_PAD,), jnp.float32),
            jax.ShapeDtypeStruct((NW, NUM_GRAPHS), jnp.float32),
        ),
        scratch_types=[
            pltpu.VMEM((CHUNK,), jnp.float32),
            pltpu.VMEM((CHUNK,), jnp.int32),
            pltpu.VMEM((CHUNK,), jnp.float32),
            pltpu.VMEM((CHUNK,), jnp.int32),
            pltpu.VMEM((CHUNK,), jnp.float32),
            pltpu.VMEM((TABLE_PAD,), jnp.float32),
            pltpu.VMEM((L * NUM_GRAPHS,), jnp.float32),
            pltpu.VMEM((NUM_GRAPHS,), jnp.float32),
        ],
    )(_sc_body)
    return k(ne_p, sp_p, lg_p, bt_p, tab_p)


def kernel(node_energy, forces, species, local_or_ghost, batch, ptr, enr_table):
    pad = N_PAD - N_NODES
    ne_p = jnp.pad(node_energy, (0, pad))
    sp_p = jnp.pad(species.astype(jnp.int32), (0, pad))
    lg_p = jnp.pad(local_or_ghost.astype(jnp.float32), (0, pad))
    bt_p = jnp.pad(batch.astype(jnp.int32), (0, pad))
    tab_p = jnp.pad(enr_table, (0, TABLE_PAD - enr_table.shape[0]))

    oe, partials = _sc_call(ne_p, sp_p, lg_p, bt_p, tab_p)

    total_energy_local = partials.sum(axis=0)
    node_energy_out = oe[:N_NODES]
    virials = jnp.zeros((1, 3, 3), dtype=node_energy.dtype)
    return (total_energy_local, node_energy_out, forces, virials)


# speedup vs baseline: 22.0072x; 1.1441x over previous
import functools

import jax
import jax.numpy as jnp
from jax import lax
from jax.experimental import pallas as pl
from jax.experimental.pallas import tpu as pltpu
from jax.experimental.pallas import tpu_sc as plsc

N_NODES = 100000
NUM_GRAPHS = 64
NW = 32
CHUNK = 3136
N_PAD = NW * CHUNK
TABLE_PAD = 128
L = 16
VECS = CHUNK // L


def _sc_body(ne_hbm, sp_hbm, lg_hbm, bt_hbm, tab_hbm,
             oe_hbm, par_hbm,
             ne_v, sp_v, lg_v, bt_v, oe_v, tab_v, acc_v, par_v, sem):
    wid = lax.axis_index("s") * 2 + lax.axis_index("c")
    base = wid * CHUNK

    sl_in = pl.ds(base, CHUNK)
    copies = [
        pltpu.async_copy(tab_hbm, tab_v, sem),
        pltpu.async_copy(ne_hbm.at[sl_in], ne_v, sem),
        pltpu.async_copy(sp_hbm.at[sl_in], sp_v, sem),
        pltpu.async_copy(lg_hbm.at[sl_in], lg_v, sem),
        pltpu.async_copy(bt_hbm.at[sl_in], bt_v, sem),
    ]

    zeros16 = jnp.zeros((L,), jnp.float32)

    @plsc.parallel_loop(0, (L * NUM_GRAPHS) // L, unroll=4)
    def _(i):
        acc_v[pl.ds(i * L, L)] = zeros16

    for c in copies:
        c.wait()

    lane_base = lax.iota(jnp.int32, L) * NUM_GRAPHS

    @plsc.parallel_loop(0, VECS, unroll=8)
    def _(i):
        sl = pl.ds(i * L, L)
        idx = sp_v[sl]
        tv = plsc.load_gather(tab_v, [idx])
        lgv = lg_v[sl]
        e = ne_v[sl] + tv * lgv
        oe_v[sl] = e
        plsc.addupdate_scatter(acc_v, [lane_base + bt_v[sl]], e * lgv)

    @plsc.parallel_loop(0, NUM_GRAPHS // L, unroll=2)
    def _(j):
        s = jnp.zeros((L,), jnp.float32)
        for l in range(L):
            s = s + acc_v[pl.ds(l * NUM_GRAPHS + j * L, L)]
        par_v[pl.ds(j * L, L)] = s

    pltpu.sync_copy(oe_v, oe_hbm.at[sl_in])
    pltpu.sync_copy(par_v, par_hbm.at[wid])


@jax.jit
def _sc_call(ne_p, sp_p, lg_p, bt_p, tab_p):
    mesh = plsc.VectorSubcoreMesh(core_axis_name="c", subcore_axis_name="s")
    k = functools.partial(
        pl.kernel,
        mesh=mesh,
        compiler_params=pltpu.CompilerParams(needs_layout_passes=False),
        out_type=(
            jax.ShapeDtypeStruct((N_PAD,), jnp.float32),
            jax.ShapeDtypeStruct((NW, NUM_GRAPHS), jnp.float32),
        ),
        scratch_types=[
            pltpu.VMEM((CHUNK,), jnp.float32),
            pltpu.VMEM((CHUNK,), jnp.int32),
            pltpu.VMEM((CHUNK,), jnp.float32),
            pltpu.VMEM((CHUNK,), jnp.int32),
            pltpu.VMEM((CHUNK,), jnp.float32),
            pltpu.VMEM((TABLE_PAD,), jnp.float32),
            pltpu.VMEM((L * NUM_GRAPHS,), jnp.float32),
            pltpu.VMEM((NUM_GRAPHS,), jnp.float32),
            pltpu.SemaphoreType.DMA,
        ],
    )(_sc_body)
    return k(ne_p, sp_p, lg_p, bt_p, tab_p)


def kernel(node_energy, forces, species, local_or_ghost, batch, ptr, enr_table):
    pad = N_PAD - N_NODES
    ne_p = jnp.pad(node_energy, (0, pad))
    sp_p = jnp.pad(species.astype(jnp.int32), (0, pad))
    lg_p = jnp.pad(local_or_ghost.astype(jnp.float32), (0, pad))
    bt_p = jnp.pad(batch.astype(jnp.int32), (0, pad))
    tab_p = jnp.pad(enr_table, (0, TABLE_PAD - enr_table.shape[0]))

    oe, partials = _sc_call(ne_p, sp_p, lg_p, bt_p, tab_p)

    total_energy_local = partials.sum(axis=0)
    node_energy_out = oe[:N_NODES]
    virials = jnp.zeros((1, 3, 3), dtype=node_energy.dtype)
    return (total_energy_local, node_energy_out, forces, virials)
